# baseline (speedup 1.0000x reference)
import functools

import jax
import jax.numpy as jnp
from jax import lax
from jax.experimental import pallas as pl
from jax.experimental.pallas import tpu as pltpu
from jax.experimental.pallas import tpu_sc as plsc

DIM = 64

_info = plsc.get_sparse_core_info()
_NC, _NS = _info.num_cores, _info.num_subcores
_NW = _NC * _NS

_CHUNK = 128
_NBUF = 4


@jax.jit
def _gather_pallas(cos_cached, sin_cached, idx):
    n = idx.shape[0]
    b_per_w = n // _NW
    n_stages = b_per_w // _CHUNK

    mesh = plsc.VectorSubcoreMesh(core_axis_name="c", subcore_axis_name="s")

    @functools.partial(
        pl.kernel,
        mesh=mesh,
        out_type=[
            jax.ShapeDtypeStruct((n, DIM), jnp.float32),
            jax.ShapeDtypeStruct((n, DIM), jnp.float32),
        ],
        scratch_types=[
            pltpu.VMEM((b_per_w,), jnp.int32),
            pltpu.VMEM((_NBUF * _CHUNK, DIM), jnp.float32),
            pltpu.VMEM((_NBUF * _CHUNK, DIM), jnp.float32),
            pltpu.SemaphoreType.DMA,
            pltpu.SemaphoreType.DMA,
        ],
        compiler_params=pltpu.CompilerParams(use_tc_tiling_on_sc=False),
    )
    def k(cos_hbm, sin_hbm, idx_hbm, cos_out, sin_out, idx_v, cos_v, sin_v, gsem, osem):
        wid = lax.axis_index("s") * _NC + lax.axis_index("c")
        base = wid * b_per_w
        pltpu.sync_copy(idx_hbm.at[pl.ds(base, b_per_w)], idx_v)

        def buf(v, s):
            return v.at[pl.ds((s % _NBUF) * _CHUNK, _CHUNK)]

        def fire(s):
            idx_c = idx_v.at[pl.ds(s * _CHUNK, _CHUNK)]
            pltpu.async_copy(cos_hbm.at[idx_c], buf(cos_v, s), gsem)
            pltpu.async_copy(sin_hbm.at[idx_c], buf(sin_v, s), gsem)

        def drain(sem, s):
            pltpu.make_async_copy(cos_hbm.at[pl.ds(0, _CHUNK)], buf(cos_v, s), sem).wait()
            pltpu.make_async_copy(sin_hbm.at[pl.ds(0, _CHUNK)], buf(sin_v, s), sem).wait()

        for s in range(min(_NBUF, n_stages)):
            fire(s)
        for s in range(n_stages):
            drain(gsem, s)
            pltpu.async_copy(buf(cos_v, s), cos_out.at[pl.ds(base + s * _CHUNK, _CHUNK)], osem)
            pltpu.async_copy(buf(sin_v, s), sin_out.at[pl.ds(base + s * _CHUNK, _CHUNK)], osem)
            if s >= 1 and s - 1 + _NBUF < n_stages:
                drain(osem, s - 1)
                fire(s - 1 + _NBUF)
        first_undrained = max(0, n_stages - _NBUF)
        for s in range(first_undrained, n_stages):
            drain(osem, s)

    return k(cos_cached, sin_cached, idx)


def kernel(x, position_ids, cos_cached, sin_cached):
    b, s = position_ids.shape
    idx = position_ids.reshape(-1).astype(jnp.int32)
    cos_flat, sin_flat = _gather_pallas(cos_cached, sin_cached, idx)
    cos = cos_flat.reshape(b, s, DIM).astype(x.dtype)
    sin = sin_flat.reshape(b, s, DIM).astype(x.dtype)
    return (cos, sin)
